# baseline (speedup 1.0000x reference)
import functools

import jax
import jax.numpy as jnp
from jax import lax
from jax.experimental import pallas as pl
from jax.experimental.pallas import tpu as pltpu
from jax.experimental.pallas import tpu_sc as plsc

N = 100000
D = 128
K = 64
BLK = 10000
NBLK = N // BLK

ROWS_PER_SUBCORE = 2
LANES = 16


def _tc_body(x_ref, ct_ref, sums_ref, cntbc_ref, cnt_ref, c2_ref):
    i = pl.program_id(0)

    @pl.when(i == 0)
    def _init():
        sums_ref[...] = jnp.zeros_like(sums_ref)
        cnt_ref[...] = jnp.zeros_like(cnt_ref)
        ct0 = ct_ref[...]
        c2_ref[...] = 0.25 * jnp.sum(ct0 * ct0, axis=0, keepdims=True)

    x = x_ref[...]
    ct = ct_ref[...]
    scores = c2_ref[...] + lax.dot_general(
        x, ct, (((1,), (0,)), ((), ())),
        preferred_element_type=jnp.float32,
    )
    m = jnp.min(scores, axis=1, keepdims=True)
    onehot = jnp.where(scores == m, 1.0, 0.0)

    sums_ref[...] += lax.dot_general(
        onehot, x, (((0,), (0,)), ((), ())),
        preferred_element_type=jnp.float32,
    )
    cnt_ref[...] += jnp.sum(onehot, axis=0, keepdims=True)

    @pl.when(i == NBLK - 1)
    def _finalize_counts():
        riota = lax.broadcasted_iota(jnp.int32, (K, K), 0)
        ciota = lax.broadcasted_iota(jnp.int32, (K, K), 1)
        eye = jnp.where(riota == ciota, 1.0, 0.0)
        cnt_col = lax.dot_general(
            eye, cnt_ref[...], (((1,), (1,)), ((), ())),
            preferred_element_type=jnp.float32,
        )
        cntbc_ref[...] = jnp.broadcast_to(cnt_col, (K, D))


def _tc_stage(features, centers_t):
    return pl.pallas_call(
        _tc_body,
        grid=(NBLK,),
        in_specs=[
            pl.BlockSpec((BLK, D), lambda i: (i, 0)),
            pl.BlockSpec((D, K), lambda i: (0, 0)),
        ],
        out_specs=[
            pl.BlockSpec((K, D), lambda i: (0, 0)),
            pl.BlockSpec((K, D), lambda i: (0, 0)),
        ],
        out_shape=[
            jax.ShapeDtypeStruct((K, D), jnp.float32),
            jax.ShapeDtypeStruct((K, D), jnp.float32),
        ],
        scratch_shapes=[
            pltpu.VMEM((1, K), jnp.float32),
            pltpu.VMEM((1, K), jnp.float32),
        ],
    )(features, centers_t)


def _sc_finalize(sums, cntbc, centers):
    mesh = plsc.VectorSubcoreMesh(core_axis_name="c", subcore_axis_name="s")

    @functools.partial(
        pl.kernel,
        mesh=mesh,
        out_type=jax.ShapeDtypeStruct((K, D), jnp.float32),
        scratch_types=[
            pltpu.VMEM((ROWS_PER_SUBCORE, D), jnp.float32),
            pltpu.VMEM((ROWS_PER_SUBCORE, D), jnp.float32),
            pltpu.VMEM((ROWS_PER_SUBCORE, D), jnp.float32),
            pltpu.VMEM((ROWS_PER_SUBCORE, D), jnp.float32),
        ],
    )
    def sc_kernel(sums_hbm, cnt_hbm, ctr_hbm, out_hbm, sums_v, cnt_v, ctr_v, out_v):
        wid = lax.axis_index("s") * 2 + lax.axis_index("c")
        base = wid * ROWS_PER_SUBCORE
        pltpu.sync_copy(sums_hbm.at[pl.ds(base, ROWS_PER_SUBCORE)], sums_v)
        pltpu.sync_copy(cnt_hbm.at[pl.ds(base, ROWS_PER_SUBCORE)], cnt_v)
        pltpu.sync_copy(ctr_hbm.at[pl.ds(base, ROWS_PER_SUBCORE)], ctr_v)
        for r in range(ROWS_PER_SUBCORE):
            for c in range(D // LANES):
                sl = (r, pl.ds(c * LANES, LANES))
                s = sums_v[sl]
                n = cnt_v[sl]
                out_v[sl] = jnp.where(n > 0.0, s / jnp.maximum(n, 1.0), ctr_v[sl])
        pltpu.sync_copy(out_v, out_hbm.at[pl.ds(base, ROWS_PER_SUBCORE)])

    return sc_kernel(sums, cntbc, centers)


@functools.partial(jax.jit)
def kernel(features, cluster_centers):
    centers_t = -2.0 * cluster_centers.T
    sums, cntbc = _tc_stage(features, centers_t)
    return _sc_finalize(sums, cntbc, cluster_centers)


# speedup vs baseline: 1.0292x; 1.0292x over previous
import functools

import jax
import jax.numpy as jnp
from jax import lax
from jax.experimental import pallas as pl
from jax.experimental.pallas import tpu as pltpu
from jax.experimental.pallas import tpu_sc as plsc

N = 100000
D = 128
K = 64
BLK = 10000
NBLK = N // BLK

ROWS_PER_SUBCORE = 4
LANES = 16


def _tc_body(x_ref, ct_ref, sums_ref, cntbc_ref, cnt_ref, c2_ref):
    i = pl.program_id(0)

    @pl.when(i == 0)
    def _init():
        sums_ref[...] = jnp.zeros_like(sums_ref)
        cnt_ref[...] = jnp.zeros_like(cnt_ref)
        ct0 = ct_ref[...]
        c2_ref[...] = 0.25 * jnp.sum(ct0 * ct0, axis=0, keepdims=True)

    x = x_ref[...]
    ct = ct_ref[...]
    scores = c2_ref[...] + lax.dot_general(
        x, ct, (((1,), (0,)), ((), ())),
        preferred_element_type=jnp.float32,
    )
    m = jnp.min(scores, axis=1, keepdims=True)
    onehot = jnp.where(scores == m, 1.0, 0.0)

    sums_ref[...] += lax.dot_general(
        onehot, x, (((0,), (0,)), ((), ())),
        preferred_element_type=jnp.float32,
    )
    cnt_ref[...] += jnp.sum(onehot, axis=0, keepdims=True)

    @pl.when(i == NBLK - 1)
    def _finalize_counts():
        riota = lax.broadcasted_iota(jnp.int32, (K, K), 0)
        ciota = lax.broadcasted_iota(jnp.int32, (K, K), 1)
        eye = jnp.where(riota == ciota, 1.0, 0.0)
        cnt_col = lax.dot_general(
            eye, cnt_ref[...], (((1,), (1,)), ((), ())),
            preferred_element_type=jnp.float32,
        )
        cntbc_ref[...] = jnp.broadcast_to(cnt_col, (K, D))


def _tc_stage(features, centers_t):
    return pl.pallas_call(
        _tc_body,
        grid=(NBLK,),
        in_specs=[
            pl.BlockSpec((BLK, D), lambda i: (i, 0)),
            pl.BlockSpec((D, K), lambda i: (0, 0)),
        ],
        out_specs=[
            pl.BlockSpec((K, D), lambda i: (0, 0)),
            pl.BlockSpec((K, D), lambda i: (0, 0)),
        ],
        out_shape=[
            jax.ShapeDtypeStruct((K, D), jnp.float32),
            jax.ShapeDtypeStruct((K, D), jnp.float32),
        ],
        scratch_shapes=[
            pltpu.VMEM((1, K), jnp.float32),
            pltpu.VMEM((1, K), jnp.float32),
        ],
    )(features, centers_t)


def _sc_finalize(sums, cntbc, centers):
    mesh = plsc.VectorSubcoreMesh(core_axis_name="c", subcore_axis_name="s", num_cores=1)

    @functools.partial(
        pl.kernel,
        mesh=mesh,
        out_type=jax.ShapeDtypeStruct((K, D), jnp.float32),
        scratch_types=[
            pltpu.VMEM((ROWS_PER_SUBCORE, D), jnp.float32),
            pltpu.VMEM((ROWS_PER_SUBCORE, D), jnp.float32),
            pltpu.VMEM((ROWS_PER_SUBCORE, D), jnp.float32),
            pltpu.VMEM((ROWS_PER_SUBCORE, D), jnp.float32),
        ],
    )
    def sc_kernel(sums_hbm, cnt_hbm, ctr_hbm, out_hbm, sums_v, cnt_v, ctr_v, out_v):
        wid = lax.axis_index("s") + lax.axis_index("c")
        base = wid * ROWS_PER_SUBCORE
        pltpu.sync_copy(sums_hbm.at[pl.ds(base, ROWS_PER_SUBCORE)], sums_v)
        pltpu.sync_copy(cnt_hbm.at[pl.ds(base, ROWS_PER_SUBCORE)], cnt_v)
        pltpu.sync_copy(ctr_hbm.at[pl.ds(base, ROWS_PER_SUBCORE)], ctr_v)
        for r in range(ROWS_PER_SUBCORE):
            for c in range(D // LANES):
                sl = (r, pl.ds(c * LANES, LANES))
                s = sums_v[sl]
                n = cnt_v[sl]
                out_v[sl] = jnp.where(n > 0.0, s / jnp.maximum(n, 1.0), ctr_v[sl])
        pltpu.sync_copy(out_v, out_hbm.at[pl.ds(base, ROWS_PER_SUBCORE)])

    return sc_kernel(sums, cntbc, centers)


@functools.partial(jax.jit)
def kernel(features, cluster_centers):
    centers_t = -2.0 * cluster_centers.T
    sums, cntbc = _tc_stage(features, centers_t)
    return _sc_finalize(sums, cntbc, cluster_centers)
